# Initial kernel scaffold; baseline (speedup 1.0000x reference)
import jax
import jax.numpy as jnp
from jax.experimental import pallas as pl


def kernel(x, table):
    raise NotImplementedError("write your pallas kernel here")


# speedup vs baseline: 1.0937x; 1.0937x over previous
import functools

import jax
import jax.numpy as jnp
from jax import lax
from jax.experimental import pallas as pl
from jax.experimental.pallas import tpu as pltpu
from jax.experimental.pallas import tpu_sc as plsc

VOCAB = 1_000_000
EMBED_DIM = 32
BATCH = 16384
HIST = 50

_NC = 2
_NS = 16
_NW = _NC * _NS

_B = BATCH * HIST
_BPW = _B // _NW
_CHUNK = 1024
_NCHUNKS = _BPW // _CHUNK

_mesh = plsc.VectorSubcoreMesh(core_axis_name="c", subcore_axis_name="s")


@functools.partial(
    pl.kernel,
    mesh=_mesh,
    out_type=jax.ShapeDtypeStruct((_B, EMBED_DIM), jnp.float32),
    scratch_types=[
        pltpu.VMEM((_CHUNK,), jnp.int32),
        pltpu.VMEM((_CHUNK, EMBED_DIM), jnp.float32),
        pltpu.SemaphoreType.DMA,
    ],
    compiler_params=pltpu.CompilerParams(use_tc_tiling_on_sc=False),
)
def _gather_kernel(idx_hbm, table_hbm, out_hbm, idx_v, rows_v, sem):
    wid = lax.axis_index("s") * _NC + lax.axis_index("c")
    base0 = wid * _BPW

    def body(g, carry):
        base = base0 + g * _CHUNK
        pltpu.sync_copy(idx_hbm.at[pl.ds(base, _CHUNK)], idx_v)
        pltpu.async_copy(table_hbm.at[idx_v], rows_v, sem).wait()
        pltpu.sync_copy(rows_v, out_hbm.at[pl.ds(base, _CHUNK)])
        return carry

    lax.fori_loop(0, _NCHUNKS, body, 0)


def kernel(x, table):
    idx = x.reshape(-1).astype(jnp.int32)
    out = _gather_kernel(idx, table)
    return out.reshape(BATCH, HIST, EMBED_DIM)


# speedup vs baseline: 1.1119x; 1.0166x over previous
import functools

import jax
import jax.numpy as jnp
from jax import lax
from jax.experimental import pallas as pl
from jax.experimental.pallas import tpu as pltpu
from jax.experimental.pallas import tpu_sc as plsc

VOCAB = 1_000_000
EMBED_DIM = 32
BATCH = 16384
HIST = 50

_NC = 2
_NS = 16
_NW = _NC * _NS

_B = BATCH * HIST
_BPW = _B // _NW
_NBUF = 4
_CHUNK = 800
_NCHUNKS = _BPW // _CHUNK
_NITER = _NCHUNKS // _NBUF

_mesh = plsc.VectorSubcoreMesh(core_axis_name="c", subcore_axis_name="s")


@functools.partial(
    pl.kernel,
    mesh=_mesh,
    out_type=jax.ShapeDtypeStruct((_B, EMBED_DIM), jnp.float32),
    scratch_types=[
        pltpu.VMEM((_BPW,), jnp.int32),
        [pltpu.VMEM((_CHUNK, EMBED_DIM), jnp.float32) for _ in range(_NBUF)],
        [pltpu.SemaphoreType.DMA for _ in range(_NBUF)],
        [pltpu.SemaphoreType.DMA for _ in range(_NBUF)],
        pltpu.SemaphoreType.DMA,
    ],
    compiler_params=pltpu.CompilerParams(use_tc_tiling_on_sc=False),
)
def _gather_kernel(idx_hbm, table_hbm, out_hbm, idx_v, rows, gsem, osem, isem):
    wid = lax.axis_index("s") * _NC + lax.axis_index("c")
    base0 = wid * _BPW

    pltpu.async_copy(idx_hbm.at[pl.ds(base0, _BPW)], idx_v, isem).wait()

    def gather_for(g, k):
        return pltpu.make_async_copy(
            table_hbm.at[idx_v.at[pl.ds(g * _CHUNK, _CHUNK)]], rows[k], gsem[k])

    def out_for(g, k):
        return pltpu.make_async_copy(
            rows[k], out_hbm.at[pl.ds(base0 + g * _CHUNK, _CHUNK)], osem[k])

    def body(i, carry):
        for k in range(_NBUF):
            g = i * _NBUF + k

            @pl.when(g >= _NBUF)
            def _():
                out_for(g, k).wait()

            gather_for(g, k).start()

            km1 = (k + _NBUF - 1) % _NBUF

            @pl.when(g >= 1)
            def _():
                gather_for(g, km1).wait()
                out_for(g - 1, km1).start()

        return carry

    lax.fori_loop(0, _NITER, body, 0)

    last = _NCHUNKS - 1
    klast = last % _NBUF
    gather_for(last, klast).wait()
    out_for(last, klast).start()
    for k in range(_NBUF):
        out_for(_NCHUNKS - _NBUF + k, k).wait()


def kernel(x, table):
    idx = x.reshape(-1).astype(jnp.int32)
    out = _gather_kernel(idx, table)
    return out.reshape(BATCH, HIST, EMBED_DIM)


# speedup vs baseline: 1.9401x; 1.7449x over previous
import functools

import jax
import jax.numpy as jnp
from jax import lax
from jax.experimental import pallas as pl
from jax.experimental.pallas import tpu as pltpu
from jax.experimental.pallas import tpu_sc as plsc

VOCAB = 1_000_000
EMBED_DIM = 32
BATCH = 16384
HIST = 50

_NC = 2
_NS = 16
_NW = _NC * _NS

_B = BATCH * HIST
_BW = BATCH // _NW
_NBUF = 4
_NCHUNKS = HIST
_NITER = 48 // _NBUF

_mesh = plsc.VectorSubcoreMesh(core_axis_name="c", subcore_axis_name="s")


@functools.partial(
    pl.kernel,
    mesh=_mesh,
    out_type=jax.ShapeDtypeStruct((_B, EMBED_DIM), jnp.float32),
    scratch_types=[
        pltpu.VMEM((HIST, _BW), jnp.int32),
        [pltpu.VMEM((_BW, EMBED_DIM), jnp.float32) for _ in range(_NBUF)],
        [pltpu.SemaphoreType.DMA for _ in range(_NBUF)],
        [pltpu.SemaphoreType.DMA for _ in range(_NBUF)],
        pltpu.SemaphoreType.DMA,
    ],
    compiler_params=pltpu.CompilerParams(use_tc_tiling_on_sc=False),
)
def _gather_kernel(idx_hbm, table_hbm, out_hbm, idx_v, rows, gsem, osem, isem):
    wid = lax.axis_index("s") * _NC + lax.axis_index("c")
    b0 = wid * _BW

    pltpu.async_copy(idx_hbm.at[:, pl.ds(b0, _BW)], idx_v, isem).wait()

    def gather_for(h, k):
        return pltpu.make_async_copy(
            table_hbm.at[idx_v.at[h]], rows[k], gsem[k])

    def out_for(h, k):
        return pltpu.make_async_copy(
            rows[k], out_hbm.at[pl.ds(h * BATCH + b0, _BW)], osem[k])

    def body(i, carry):
        for k in range(_NBUF):
            h = i * _NBUF + k

            @pl.when(h >= _NBUF)
            def _():
                out_for(h, k).wait()

            gather_for(h, k).start()

            km1 = (k + _NBUF - 1) % _NBUF

            @pl.when(h >= 1)
            def _():
                gather_for(h, km1).wait()
                out_for(h - 1, km1).start()

        return carry

    lax.fori_loop(0, _NITER, body, 0)

    for h in (48, 49):
        k = h % _NBUF
        out_for(h, k).wait()
        gather_for(h, k).start()
        km1 = (k + _NBUF - 1) % _NBUF
        gather_for(h, km1).wait()
        out_for(h - 1, km1).start()
    gather_for(49, 49 % _NBUF).wait()
    out_for(49, 49 % _NBUF).start()
    for h in range(_NCHUNKS - _NBUF, _NCHUNKS):
        out_for(h, h % _NBUF).wait()


def kernel(x, table):
    xt = x.T.astype(jnp.int32)
    out = _gather_kernel(xt, table)
    return out.reshape(HIST, BATCH, EMBED_DIM).transpose(1, 0, 2)


# speedup vs baseline: 1.9420x; 1.0010x over previous
import functools

import jax
import jax.numpy as jnp
from jax import lax
from jax.experimental import pallas as pl
from jax.experimental.pallas import tpu as pltpu
from jax.experimental.pallas import tpu_sc as plsc

VOCAB = 1_000_000
EMBED_DIM = 32
BATCH = 16384
HIST = 50

_NC = 2
_NS = 16
_NW = _NC * _NS

_B = BATCH * HIST
_BW = BATCH // _NW
_NBUF = 4
_NCHUNKS = HIST
_NITER = 48 // _NBUF

_mesh = plsc.VectorSubcoreMesh(core_axis_name="c", subcore_axis_name="s")


@functools.partial(
    pl.kernel,
    mesh=_mesh,
    out_type=jax.ShapeDtypeStruct((_B, EMBED_DIM), jnp.float32),
    scratch_types=[
        pltpu.VMEM((HIST, _BW), jnp.int32),
        [pltpu.VMEM((_BW, EMBED_DIM), jnp.float32) for _ in range(_NBUF)],
        [pltpu.SemaphoreType.DMA for _ in range(_NBUF)],
        [pltpu.SemaphoreType.DMA for _ in range(_NBUF)],
        pltpu.SemaphoreType.DMA,
    ],
    compiler_params=pltpu.CompilerParams(use_tc_tiling_on_sc=False),
)
def _gather_kernel(idx_hbm, table_hbm, out_hbm, idx_v, rows, gsem, osem, isem):
    wid = lax.axis_index("s") * _NC + lax.axis_index("c")
    b0 = wid * _BW

    pltpu.async_copy(idx_hbm.at[:, pl.ds(b0, _BW)], idx_v, isem).wait()

    def gather_for(h, k):
        return pltpu.make_async_copy(
            table_hbm.at[idx_v.at[h]], rows[k], gsem[k])

    def out_for(h, k):
        return pltpu.make_async_copy(
            rows[k], out_hbm.at[pl.ds(h * BATCH + b0, _BW)], osem[k])

    def body(i, carry):
        for k in range(_NBUF):
            h = i * _NBUF + k

            @pl.when(h >= _NBUF)
            def _():
                out_for(h, k).wait()

            gather_for(h, k).start()

            km1 = (k + _NBUF - 1) % _NBUF

            @pl.when(h >= 1)
            def _():
                gather_for(h, km1).wait()
                out_for(h - 1, km1).start()

        return carry

    lax.fori_loop(0, _NITER, body, 0)

    for h in (48, 49):
        k = h % _NBUF
        out_for(h, k).wait()
        gather_for(h, k).start()
        km1 = (k + _NBUF - 1) % _NBUF
        gather_for(h, km1).wait()
        out_for(h - 1, km1).start()
    gather_for(49, 49 % _NBUF).wait()
    out_for(49, 49 % _NBUF).start()
    for h in range(_NCHUNKS - _NBUF, _NCHUNKS):
        out_for(h, h % _NBUF).wait()


def kernel(x, table):
    xt = x.T.astype(jnp.int32)
    out = _gather_kernel(xt, table)
    out = out.reshape(_B // 4, 4 * EMBED_DIM)
    return out.reshape(HIST, BATCH, EMBED_DIM).transpose(1, 0, 2)
